# baseline (device time: 24718 ns/iter reference)
import jax
import jax.numpy as jnp
from jax import lax
from jax.experimental import pallas as pl
from jax.experimental.pallas import tpu as pltpu

N_DEV = 4
N_ROWS = 512
D_IN = 256
D_OUT = 512
N_EXPERTS = 16
EXP_PER_DEV = N_EXPERTS // N_DEV
ROWS_PER_DEV = N_ROWS // N_DEV


def kernel(x, router_W, route_idx, expert_W, shared_W):
    def body(x_ref, rw_ref, ridx_ref, ew_ref, sw_ref, out_ref,
             part_ref, comm_ref, send_sems, recv_sems):
        my = lax.axis_index("i")
        left = lax.rem(my + N_DEV - 1, N_DEV)
        right = lax.rem(my + 1, N_DEV)

        barrier_sem = pltpu.get_barrier_semaphore()
        for nbr in (left, right):
            pl.semaphore_signal(
                barrier_sem, inc=1,
                device_id=(nbr,), device_id_type=pl.DeviceIdType.MESH,
            )
        pl.semaphore_wait(barrier_sem, 2)

        xv = x_ref[:, :]
        scores = jnp.dot(xv, rw_ref[:, :], preferred_element_type=jnp.float32)
        s_max = jnp.max(scores, axis=1, keepdims=True)
        e = jnp.exp(scores - s_max)
        probs = e / jnp.sum(e, axis=1, keepdims=True)
        idx = ridx_ref[:, :]
        onehot = lax.broadcasted_iota(jnp.int32, (N_ROWS, N_EXPERTS), 1) == idx
        p_sel = jnp.sum(jnp.where(onehot, probs, 0.0), axis=1, keepdims=True)

        acc = jnp.zeros((N_ROWS, D_OUT), jnp.float32)
        for le in range(EXP_PER_DEV):
            e_glob = my * EXP_PER_DEV + le
            w = jnp.where(idx == e_glob, p_sel, 0.0)
            acc = acc + jnp.dot(xv * w, ew_ref[le, :, :],
                                preferred_element_type=jnp.float32)
        part_ref[:, :] = acc

        c0 = lax.rem(my + N_DEV - 1, N_DEV)
        comm_ref[0, :, :] = part_ref[pl.ds(c0 * ROWS_PER_DEV, ROWS_PER_DEV), :]
        for h in range(N_DEV - 1):
            rdma = pltpu.make_async_remote_copy(
                src_ref=comm_ref.at[h],
                dst_ref=comm_ref.at[h + 1],
                send_sem=send_sems.at[h],
                recv_sem=recv_sems.at[h],
                device_id=(right,),
                device_id_type=pl.DeviceIdType.MESH,
            )
            rdma.start()
            rdma.wait()
            c = lax.rem(my + 2 * N_DEV - 2 - h, N_DEV)
            comm_ref[h + 1, :, :] = (
                comm_ref[h + 1, :, :]
                + part_ref[pl.ds(c * ROWS_PER_DEV, ROWS_PER_DEV), :]
            )

        x_blk = x_ref[pl.ds(my * ROWS_PER_DEV, ROWS_PER_DEV), :]
        shared = jnp.dot(x_blk, sw_ref[:, :], preferred_element_type=jnp.float32)
        out_ref[:, :] = comm_ref[N_DEV - 1, :, :] + shared

    return pl.pallas_call(
        body,
        out_shape=jax.ShapeDtypeStruct((ROWS_PER_DEV, D_OUT), jnp.float32),
        in_specs=[pl.BlockSpec(memory_space=pltpu.VMEM)] * 5,
        out_specs=pl.BlockSpec(memory_space=pltpu.VMEM),
        scratch_shapes=[
            pltpu.VMEM((N_ROWS, D_OUT), jnp.float32),
            pltpu.VMEM((N_DEV, ROWS_PER_DEV, D_OUT), jnp.float32),
            pltpu.SemaphoreType.DMA((N_DEV - 1,)),
            pltpu.SemaphoreType.DMA((N_DEV - 1,)),
        ],
        compiler_params=pltpu.CompilerParams(collective_id=0),
    )(x, router_W, route_idx, expert_W, shared_W)


# device time: 14266 ns/iter; 1.7327x vs baseline; 1.7327x over previous
import jax
import jax.numpy as jnp
from jax import lax
from jax.experimental import pallas as pl
from jax.experimental.pallas import tpu as pltpu

N_DEV = 4
N_ROWS = 512
D_IN = 256
D_OUT = 512
N_EXPERTS = 16
EXP_PER_DEV = N_EXPERTS // N_DEV
ROWS_PER_DEV = N_ROWS // N_DEV


def kernel(x, router_W, route_idx, expert_W, shared_W):
    def body(x_ref, rw_ref, ridx_ref, ew_ref, sw_ref, out_ref,
             part_ref, send_ref, recv_ref, send_sems, recv_sems):
        my = lax.axis_index("i")

        barrier_sem = pltpu.get_barrier_semaphore()
        for k in range(1, N_DEV):
            pl.semaphore_signal(
                barrier_sem, inc=1,
                device_id=(lax.rem(my + k, N_DEV),),
                device_id_type=pl.DeviceIdType.MESH,
            )

        xv = x_ref[:, :]
        scores = jnp.dot(xv, rw_ref[:, :], preferred_element_type=jnp.float32)
        s_max = jnp.max(scores, axis=1, keepdims=True)
        e = jnp.exp(scores - s_max)
        probs = e / jnp.sum(e, axis=1, keepdims=True)
        idx = ridx_ref[:, :]
        onehot = lax.broadcasted_iota(jnp.int32, (N_ROWS, N_EXPERTS), 1) == idx
        p_sel = jnp.sum(jnp.where(onehot, probs, 0.0), axis=1, keepdims=True)

        xp = xv * p_sel
        blocks = []
        for le in range(EXP_PER_DEV):
            keep = idx == my * EXP_PER_DEV + le
            blocks.append(jnp.where(keep, xp, 0.0).astype(jnp.bfloat16))
        x_big = jnp.concatenate(blocks, axis=1)
        ew_stack = ew_ref[:, :, :].astype(jnp.bfloat16).reshape(
            EXP_PER_DEV * D_IN, D_OUT)
        part_ref[:, :] = jnp.dot(x_big, ew_stack,
                                 preferred_element_type=jnp.float32)

        for k in (2, 1, 3):
            dst = lax.rem(my + k, N_DEV)
            send_ref[k - 1, :, :] = part_ref[
                pl.ds(dst * ROWS_PER_DEV, ROWS_PER_DEV), :
            ].astype(jnp.bfloat16)

        pl.semaphore_wait(barrier_sem, N_DEV - 1)
        rdmas = {}
        for k in (2, 1, 3):
            dst = lax.rem(my + k, N_DEV)
            rdma = pltpu.make_async_remote_copy(
                src_ref=send_ref.at[k - 1],
                dst_ref=recv_ref.at[k - 1],
                send_sem=send_sems.at[k - 1],
                recv_sem=recv_sems.at[k - 1],
                device_id=(dst,),
                device_id_type=pl.DeviceIdType.MESH,
            )
            rdma.start()
            rdmas[k] = rdma

        x_blk = x_ref[pl.ds(my * ROWS_PER_DEV, ROWS_PER_DEV), :]
        shared = jnp.dot(x_blk.astype(jnp.bfloat16),
                         sw_ref[:, :].astype(jnp.bfloat16),
                         preferred_element_type=jnp.float32)
        acc = part_ref[pl.ds(my * ROWS_PER_DEV, ROWS_PER_DEV), :] + shared

        for k in range(1, N_DEV):
            rdmas[k].wait_recv()
            acc = acc + recv_ref[k - 1, :, :].astype(jnp.float32)
        out_ref[:, :] = acc

        for k in range(1, N_DEV):
            rdmas[k].wait_send()

    return pl.pallas_call(
        body,
        out_shape=jax.ShapeDtypeStruct((ROWS_PER_DEV, D_OUT), jnp.float32),
        in_specs=[pl.BlockSpec(memory_space=pltpu.VMEM)] * 5,
        out_specs=pl.BlockSpec(memory_space=pltpu.VMEM),
        scratch_shapes=[
            pltpu.VMEM((N_ROWS, D_OUT), jnp.float32),
            pltpu.VMEM((N_DEV - 1, ROWS_PER_DEV, D_OUT), jnp.bfloat16),
            pltpu.VMEM((N_DEV - 1, ROWS_PER_DEV, D_OUT), jnp.bfloat16),
            pltpu.SemaphoreType.DMA((N_DEV - 1,)),
            pltpu.SemaphoreType.DMA((N_DEV - 1,)),
        ],
        compiler_params=pltpu.CompilerParams(collective_id=0),
    )(x, router_W, route_idx, expert_W, shared_W)
